# baseline (speedup 1.0000x reference)
import functools

import jax
import jax.numpy as jnp
from jax import lax
from jax.experimental import pallas as pl
from jax.experimental.pallas import tpu as pltpu
from jax.experimental.pallas import tpu_sc as plsc

N = 100000
D = 16
L = 100000
O = 64
C = 1

NC = 2
NS = 16

CHUNK = 128
NCHUNK = 50
IDXR = 56
PW = CHUNK * NCHUNK
NPAD = PW * NS

LPAD = 102400
NPASS = 5
RANGE = LPAD // (NC * NPASS)
DUMP = CHUNK
STRIPE = RANGE // NS
ZROWS = STRIPE
NGROUP = 5
GSZ = NCHUNK // NGROUP

RB = 512
KSTEPS = LPAD // (8 * RB)
WROWS = (O * L) // 512


def _sc_segment_sum(x_w, idx_w):
    mesh = plsc.VectorSubcoreMesh(
        core_axis_name="c", subcore_axis_name="s",
        num_cores=NC, num_subcores=NS)

    @functools.partial(
        pl.kernel,
        out_type=jax.ShapeDtypeStruct((LPAD, D), jnp.float32),
        mesh=mesh,
        scratch_types=[
            pltpu.VMEM((NCHUNK, CHUNK), jnp.int32),
            pltpu.VMEM((NCHUNK, CHUNK, D), jnp.float32),
            pltpu.VMEM((ZROWS, D), jnp.float32),
            pltpu.VMEM_SHARED((RANGE + DUMP, D), jnp.float32),
            pltpu.SemaphoreType.DMA,
        ],
        compiler_params=pltpu.CompilerParams(use_tc_tiling_on_sc=False),
    )
    def k(x_hbm, idx_hbm, out_hbm, idx_v, x_v, z_v, s_sh, sem):
        c = lax.axis_index("c")
        s = lax.axis_index("s")
        zvec = jnp.zeros((D,), jnp.float32)

        @pl.loop(0, ZROWS)
        def _zfill(i):
            z_v[i, :] = zvec

        pltpu.sync_copy(x_hbm.at[s], x_v)

        for p in range(NPASS):
            blk = c * NPASS + p
            for z in range(STRIPE // ZROWS):
                pltpu.sync_copy(
                    z_v, s_sh.at[pl.ds(s * STRIPE + z * ZROWS, ZROWS)])
            pltpu.sync_copy(idx_hbm.at[c, p, s, pl.ds(0, NCHUNK)], idx_v)
            plsc.subcore_barrier()

            for g in range(NGROUP):
                @pl.loop(g * GSZ, (g + 1) * GSZ)
                def _fire(j):
                    pltpu.async_copy(x_v.at[j], s_sh.at[idx_v.at[j]], sem,
                                     add=True)

                @pl.loop(g * GSZ, (g + 1) * GSZ)
                def _drain(j):
                    pltpu.make_async_copy(x_v.at[j], s_sh.at[idx_v.at[j]],
                                          sem).wait()

            plsc.subcore_barrier()
            pltpu.sync_copy(
                s_sh.at[pl.ds(s * STRIPE, STRIPE)],
                out_hbm.at[pl.ds(blk * RANGE + s * STRIPE, STRIPE)])

    return k(x_w, idx_w)


def _tc_matmul_body(w_ref, s_ref, o_ref):
    kstep = pl.program_id(0)

    @pl.when(kstep == 0)
    def _():
        o_ref[...] = jnp.zeros_like(o_ref)

    w = w_ref[...]
    row = lax.broadcasted_iota(jnp.int32, (RB, 1), 0) + kstep * RB
    w = jnp.where(row < WROWS, w, 0.0)
    s = s_ref[...]
    acc = jnp.zeros((O, D), jnp.float32)
    for j in range(8):
        wj = w[:, 64 * j:64 * (j + 1)]
        sj = s[:, 16 * j:16 * (j + 1)]
        acc += lax.dot_general(wj, sj, (((0,), (0,)), ((), ())),
                               preferred_element_type=jnp.float32)
    o_ref[...] += acc


def kernel(x, node_labels, Param_W, Param_b):
    x = x.astype(jnp.float32)
    labels = node_labels.astype(jnp.int32)

    x_p = jnp.pad(x, ((0, NPAD - N), (0, 0)))
    lab_p = jnp.pad(labels, (0, NPAD - N), constant_values=-1)
    pos = jnp.arange(NPAD, dtype=jnp.int32) % DUMP
    blk = lab_p // RANGE
    rel = lab_p - blk * RANGE
    tgt = jnp.arange(NC * NPASS, dtype=jnp.int32)[:, None]
    idx_w = jnp.where(blk[None, :] == tgt, rel[None, :],
                      (RANGE + pos)[None, :])
    idx_w = idx_w.reshape(NC, NPASS, NS, NCHUNK, CHUNK)
    idx_w = jnp.pad(idx_w, ((0, 0), (0, 0), (0, 0), (0, IDXR - NCHUNK),
                            (0, 0)))

    x_w = x_p.reshape(NS, NCHUNK, CHUNK, D)

    s_tab = _sc_segment_sum(x_w, idx_w)
    s128 = s_tab.reshape(RB * KSTEPS, 128)

    wflat = Param_W.reshape(O, WROWS, 8).transpose(1, 2, 0).reshape(
        WROWS, 512)

    out = pl.pallas_call(
        _tc_matmul_body,
        grid=(KSTEPS,),
        in_specs=[
            pl.BlockSpec((RB, 512), lambda k: (k, 0)),
            pl.BlockSpec((RB, 128), lambda k: (k, 0)),
        ],
        out_specs=pl.BlockSpec((O, D), lambda k: (0, 0)),
        out_shape=jax.ShapeDtypeStruct((O, D), jnp.float32),
        compiler_params=pltpu.CompilerParams(
            dimension_semantics=("arbitrary",)),
    )(wflat, s128)

    return (out + Param_b.reshape(O, D)).reshape(C, O, D)


# speedup vs baseline: 3.2572x; 3.2572x over previous
import functools

import jax
import jax.numpy as jnp
from jax import lax
from jax.experimental import pallas as pl
from jax.experimental.pallas import tpu as pltpu
from jax.experimental.pallas import tpu_sc as plsc

N = 100000
D = 16
L = 100000
O = 64
C = 1

NC = 2
NS = 16

CHUNK = 128
NCHUNK = 50
IDXR = 56
PW = CHUNK * NCHUNK
NPAD = PW * NS

LPAD = 102400
NPASS = 5
RANGE = LPAD // (NC * NPASS)
DUMP = CHUNK
STRIPE = RANGE // NS
ZROWS = STRIPE
NGROUP = 5
GSZ = NCHUNK // NGROUP

LB = 10240
KSTEPS = LPAD // LB


def _sc_segment_sum(x_w, idx_w):
    mesh = plsc.VectorSubcoreMesh(
        core_axis_name="c", subcore_axis_name="s",
        num_cores=NC, num_subcores=NS)

    @functools.partial(
        pl.kernel,
        out_type=jax.ShapeDtypeStruct((LPAD, D), jnp.float32),
        mesh=mesh,
        scratch_types=[
            pltpu.VMEM((NCHUNK, CHUNK), jnp.int32),
            pltpu.VMEM((NCHUNK, CHUNK, D), jnp.float32),
            pltpu.VMEM((ZROWS, D), jnp.float32),
            pltpu.VMEM_SHARED((RANGE + DUMP, D), jnp.float32),
            pltpu.SemaphoreType.DMA,
        ],
        compiler_params=pltpu.CompilerParams(use_tc_tiling_on_sc=False),
    )
    def k(x_hbm, idx_hbm, out_hbm, idx_v, x_v, z_v, s_sh, sem):
        c = lax.axis_index("c")
        s = lax.axis_index("s")
        zvec = jnp.zeros((D,), jnp.float32)

        @pl.loop(0, ZROWS)
        def _zfill(i):
            z_v[i, :] = zvec

        pltpu.sync_copy(x_hbm.at[s], x_v)

        for p in range(NPASS):
            blk = c * NPASS + p
            for z in range(STRIPE // ZROWS):
                pltpu.sync_copy(
                    z_v, s_sh.at[pl.ds(s * STRIPE + z * ZROWS, ZROWS)])
            pltpu.sync_copy(idx_hbm.at[c, p, s, pl.ds(0, NCHUNK)], idx_v)
            plsc.subcore_barrier()

            for g in range(NGROUP):
                @pl.loop(g * GSZ, (g + 1) * GSZ)
                def _fire(j):
                    pltpu.async_copy(x_v.at[j], s_sh.at[idx_v.at[j]], sem,
                                     add=True)

                @pl.loop(g * GSZ, (g + 1) * GSZ)
                def _drain(j):
                    pltpu.make_async_copy(x_v.at[j], s_sh.at[idx_v.at[j]],
                                          sem).wait()

            plsc.subcore_barrier()
            pltpu.sync_copy(
                s_sh.at[pl.ds(s * STRIPE, STRIPE)],
                out_hbm.at[pl.ds(blk * RANGE + s * STRIPE, STRIPE)])

    return k(x_w, idx_w)


def _tc_matmul_body(w_ref, s_ref, o_ref):
    kstep = pl.program_id(0)

    @pl.when(kstep == 0)
    def _():
        o_ref[...] = jnp.zeros_like(o_ref)

    w = w_ref[...]
    col = lax.broadcasted_iota(jnp.int32, (1, LB), 1) + kstep * LB
    w = jnp.where(col < L, w, 0.0)
    o_ref[...] += jnp.dot(w, s_ref[...], preferred_element_type=jnp.float32)


def kernel(x, node_labels, Param_W, Param_b):
    x = x.astype(jnp.float32)
    labels = node_labels.astype(jnp.int32)

    x_p = jnp.pad(x, ((0, NPAD - N), (0, 0)))
    lab_p = jnp.pad(labels, (0, NPAD - N), constant_values=-1)
    pos = jnp.arange(NPAD, dtype=jnp.int32) % DUMP
    blk = lab_p // RANGE
    rel = lab_p - blk * RANGE
    tgt = jnp.arange(NC * NPASS, dtype=jnp.int32)[:, None]
    idx_w = jnp.where(blk[None, :] == tgt, rel[None, :],
                      (RANGE + pos)[None, :])
    idx_w = idx_w.reshape(NC, NPASS, NS, NCHUNK, CHUNK)
    idx_w = jnp.pad(idx_w, ((0, 0), (0, 0), (0, 0), (0, IDXR - NCHUNK),
                            (0, 0)))

    x_w = x_p.reshape(NS, NCHUNK, CHUNK, D)

    s_tab = _sc_segment_sum(x_w, idx_w)

    w2 = Param_W.reshape(O, L)

    out = pl.pallas_call(
        _tc_matmul_body,
        grid=(KSTEPS,),
        in_specs=[
            pl.BlockSpec((O, LB), lambda k: (0, k)),
            pl.BlockSpec((LB, D), lambda k: (k, 0)),
        ],
        out_specs=pl.BlockSpec((O, D), lambda k: (0, 0)),
        out_shape=jax.ShapeDtypeStruct((O, D), jnp.float32),
        compiler_params=pltpu.CompilerParams(
            dimension_semantics=("arbitrary",)),
    )(w2, s_tab)

    return (out + Param_b.reshape(O, D)).reshape(C, O, D)


# speedup vs baseline: 3.6778x; 1.1291x over previous
import functools

import jax
import jax.numpy as jnp
from jax import lax
from jax.experimental import pallas as pl
from jax.experimental.pallas import tpu as pltpu
from jax.experimental.pallas import tpu_sc as plsc

N = 100000
D = 16
L = 100000
O = 64
C = 1

NC = 2
NS = 16

CHUNK = 128
PW = N // NS
NCHUNK = 49
TAIL = PW - CHUNK * (NCHUNK - 1)
TBASE = PW - CHUNK
TDUP = CHUNK - TAIL
IDXR = 56

LPAD = 102400
NPASS = 5
RANGE = LPAD // (NC * NPASS)
DUMP = CHUNK
STRIPE = RANGE // NS
ZROWS = STRIPE
NGROUP = 7
GSZ = NCHUNK // NGROUP

LB = 10240
KSTEPS = LPAD // LB


def _sc_segment_sum(x_w, idx_w):
    mesh = plsc.VectorSubcoreMesh(
        core_axis_name="c", subcore_axis_name="s",
        num_cores=NC, num_subcores=NS)

    @functools.partial(
        pl.kernel,
        out_type=jax.ShapeDtypeStruct((LPAD, D), jnp.float32),
        mesh=mesh,
        scratch_types=[
            pltpu.VMEM((NCHUNK, CHUNK), jnp.int32),
            pltpu.VMEM((PW, D), jnp.float32),
            pltpu.VMEM((ZROWS, D), jnp.float32),
            pltpu.VMEM_SHARED((RANGE + DUMP, D), jnp.float32),
            pltpu.SemaphoreType.DMA,
        ],
        compiler_params=pltpu.CompilerParams(use_tc_tiling_on_sc=False),
    )
    def k(x_hbm, idx_hbm, out_hbm, idx_v, x_v, z_v, s_sh, sem):
        c = lax.axis_index("c")
        s = lax.axis_index("s")
        zvec = jnp.zeros((D,), jnp.float32)

        @pl.loop(0, ZROWS)
        def _zfill(i):
            z_v[i, :] = zvec

        pltpu.sync_copy(x_hbm.at[s], x_v)

        def chunk_src(j):
            base = jnp.where(j == NCHUNK - 1, TBASE, j * CHUNK)
            return x_v.at[pl.ds(base, CHUNK)]

        for p in range(NPASS):
            blk = c * NPASS + p
            pltpu.sync_copy(z_v, s_sh.at[pl.ds(s * STRIPE, STRIPE)])
            pltpu.sync_copy(idx_hbm.at[c, p, s, pl.ds(0, NCHUNK)], idx_v)
            plsc.subcore_barrier()

            for g in range(NGROUP):
                @pl.loop(g * GSZ, (g + 1) * GSZ)
                def _fire(j):
                    pltpu.async_copy(chunk_src(j), s_sh.at[idx_v.at[j]],
                                     sem, add=True)

                @pl.loop(g * GSZ, (g + 1) * GSZ)
                def _drain(j):
                    pltpu.make_async_copy(chunk_src(j), s_sh.at[idx_v.at[j]],
                                          sem).wait()

            plsc.subcore_barrier()
            pltpu.sync_copy(
                s_sh.at[pl.ds(s * STRIPE, STRIPE)],
                out_hbm.at[pl.ds(blk * RANGE + s * STRIPE, STRIPE)])

    return k(x_w, idx_w)


def _tc_matmul_body(w_ref, s_ref, o_ref):
    kstep = pl.program_id(0)

    @pl.when(kstep == 0)
    def _():
        o_ref[...] = jnp.zeros_like(o_ref)

    w = w_ref[...]
    col = lax.broadcasted_iota(jnp.int32, (1, LB), 1) + kstep * LB
    w = jnp.where(col < L, w, 0.0)
    o_ref[...] += jnp.dot(w, s_ref[...], preferred_element_type=jnp.float32)


def kernel(x, node_labels, Param_W, Param_b):
    x = x.astype(jnp.float32)
    labels = node_labels.astype(jnp.int32)

    lab_w = labels.reshape(NS, PW)
    full = lab_w[:, :CHUNK * (NCHUNK - 1)].reshape(NS, NCHUNK - 1, CHUNK)
    tail = lab_w[:, TBASE:].reshape(NS, 1, CHUNK)
    lab_c = jnp.concatenate([full, tail], axis=1)
    tpos = jnp.arange(CHUNK, dtype=jnp.int32)
    tmask = (jnp.arange(NCHUNK)[:, None] == NCHUNK - 1) & (tpos[None, :] < TDUP)
    lab_c = jnp.where(tmask[None, :, :], -1, lab_c)

    pos = jnp.arange(NCHUNK * CHUNK, dtype=jnp.int32).reshape(
        NCHUNK, CHUNK) % DUMP
    blk = lab_c // RANGE
    rel = lab_c - blk * RANGE
    tgt = jnp.arange(NC * NPASS, dtype=jnp.int32)[:, None, None, None]
    idx_w = jnp.where(blk[None] == tgt, rel[None],
                      (RANGE + pos)[None, None])
    idx_w = idx_w.reshape(NC, NPASS, NS, NCHUNK, CHUNK)
    idx_w = jnp.pad(idx_w, ((0, 0), (0, 0), (0, 0), (0, IDXR - NCHUNK),
                            (0, 0)))

    x_w = x.reshape(NS, PW, D)

    s_tab = _sc_segment_sum(x_w, idx_w)

    w2 = Param_W.reshape(O, L)

    out = pl.pallas_call(
        _tc_matmul_body,
        grid=(KSTEPS,),
        in_specs=[
            pl.BlockSpec((O, LB), lambda k: (0, k)),
            pl.BlockSpec((LB, D), lambda k: (k, 0)),
        ],
        out_specs=pl.BlockSpec((O, D), lambda k: (0, 0)),
        out_shape=jax.ShapeDtypeStruct((O, D), jnp.float32),
        compiler_params=pltpu.CompilerParams(
            dimension_semantics=("arbitrary",)),
    )(w2, s_tab)

    return (out + Param_b.reshape(O, D)).reshape(C, O, D)


# speedup vs baseline: 3.9295x; 1.0684x over previous
import functools

import jax
import jax.numpy as jnp
from jax import lax
from jax.experimental import pallas as pl
from jax.experimental.pallas import tpu as pltpu
from jax.experimental.pallas import tpu_sc as plsc

N = 100000
D = 16
L = 100000
O = 64
C = 1

NC = 2
NS = 16
NLANE = 16

CHUNK = 128
PW = N // NS
NCHUNK = 49
PWPAD = NCHUNK * CHUNK

LPAD = 101376
NPASS = 3
RANGE = LPAD // (NC * NPASS)
DUMP = CHUNK
STRIPE = RANGE // NS
ZROWS = 66
NGROUP = 7
GSZ = NCHUNK // NGROUP

LB = 9216
KSTEPS = LPAD // LB


def _sc_segment_sum(x_w, lab_w):
    mesh = plsc.VectorSubcoreMesh(
        core_axis_name="c", subcore_axis_name="s",
        num_cores=NC, num_subcores=NS)

    @functools.partial(
        pl.kernel,
        out_type=jax.ShapeDtypeStruct((LPAD, D), jnp.float32),
        mesh=mesh,
        scratch_types=[
            pltpu.VMEM((PWPAD,), jnp.int32),
            pltpu.VMEM((NCHUNK, CHUNK), jnp.int32),
            pltpu.VMEM((PWPAD, D), jnp.float32),
            pltpu.VMEM((ZROWS, D), jnp.float32),
            pltpu.VMEM_SHARED((RANGE + DUMP, D), jnp.float32),
            pltpu.SemaphoreType.DMA,
        ],
        compiler_params=pltpu.CompilerParams(use_tc_tiling_on_sc=False),
    )
    def k(x_hbm, lab_hbm, out_hbm, lab_v, idx_v, x_v, z_v, s_sh, sem):
        c = lax.axis_index("c")
        s = lax.axis_index("s")
        zvec = jnp.zeros((D,), jnp.float32)

        @pl.loop(0, ZROWS)
        def _zfill(i):
            z_v[i, :] = zvec

        pltpu.sync_copy(lab_hbm.at[s], lab_v)
        pltpu.sync_copy(x_hbm.at[s], x_v.at[pl.ds(0, PW)])

        lane = lax.iota(jnp.int32, NLANE)

        for p in range(NPASS):
            blk = c * NPASS + p
            base = blk * RANGE
            for z in range(STRIPE // ZROWS):
                pltpu.sync_copy(
                    z_v, s_sh.at[pl.ds(s * STRIPE + z * ZROWS, ZROWS)])

            @pl.loop(0, NCHUNK)
            def _mkidx(j):
                for t in range(CHUNK // NLANE):
                    lab = lab_v[pl.ds(j * CHUNK + t * NLANE, NLANE)]
                    rel = lab - base
                    inr = (rel >= 0) & (rel < RANGE)
                    dump = (RANGE + t * NLANE) + lane
                    idx_v[j, pl.ds(t * NLANE, NLANE)] = jnp.where(
                        inr, rel, dump)

            plsc.subcore_barrier()

            for g in range(NGROUP):
                @pl.loop(g * GSZ, (g + 1) * GSZ)
                def _fire(j):
                    pltpu.async_copy(x_v.at[pl.ds(j * CHUNK, CHUNK)],
                                     s_sh.at[idx_v.at[j]], sem, add=True)

                @pl.loop(g * GSZ, (g + 1) * GSZ)
                def _drain(j):
                    pltpu.make_async_copy(x_v.at[pl.ds(j * CHUNK, CHUNK)],
                                          s_sh.at[idx_v.at[j]], sem).wait()

            plsc.subcore_barrier()
            pltpu.sync_copy(
                s_sh.at[pl.ds(s * STRIPE, STRIPE)],
                out_hbm.at[pl.ds(base + s * STRIPE, STRIPE)])

    return k(x_w, lab_w)


def _tc_matmul_body(w_ref, s_ref, o_ref):
    kstep = pl.program_id(0)

    @pl.when(kstep == 0)
    def _():
        o_ref[...] = jnp.zeros_like(o_ref)

    w = w_ref[...]
    col = lax.broadcasted_iota(jnp.int32, (1, LB), 1) + kstep * LB
    w = jnp.where(col < L, w, 0.0)
    o_ref[...] += jnp.dot(w, s_ref[...], preferred_element_type=jnp.float32)


def kernel(x, node_labels, Param_W, Param_b):
    x = x.astype(jnp.float32)
    labels = node_labels.astype(jnp.int32)

    lab_w = jnp.pad(labels.reshape(NS, PW), ((0, 0), (0, PWPAD - PW)),
                    constant_values=-1)
    x_w = x.reshape(NS, PW, D)

    s_tab = _sc_segment_sum(x_w, lab_w)

    w2 = Param_W.reshape(O, L)

    out = pl.pallas_call(
        _tc_matmul_body,
        grid=(KSTEPS,),
        in_specs=[
            pl.BlockSpec((O, LB), lambda k: (0, k)),
            pl.BlockSpec((LB, D), lambda k: (k, 0)),
        ],
        out_specs=pl.BlockSpec((O, D), lambda k: (0, 0)),
        out_shape=jax.ShapeDtypeStruct((O, D), jnp.float32),
        compiler_params=pltpu.CompilerParams(
            dimension_semantics=("arbitrary",)),
    )(w2, s_tab)

    return (out + Param_b.reshape(O, D)).reshape(C, O, D)


# speedup vs baseline: 4.1782x; 1.0633x over previous
import functools

import jax
import jax.numpy as jnp
from jax import lax
from jax.experimental import pallas as pl
from jax.experimental.pallas import tpu as pltpu
from jax.experimental.pallas import tpu_sc as plsc

N = 100000
D = 16
L = 100000
O = 64
C = 1

NC = 2
NS = 16
NLANE = 16

CHUNK = 128
PW = N // NS
NCHUNK = 49
PWPAD = NCHUNK * CHUNK

LPAD = 101376
NPASS = 3
RANGE = LPAD // (NC * NPASS)
DUMP = CHUNK
STRIPE = RANGE // NS
ZROWS = 66
NGROUP = 7
GSZ = NCHUNK // NGROUP

LB = 9216
KSTEPS = LPAD // LB


def _sc_segment_sum(x_w, lab_w):
    mesh = plsc.VectorSubcoreMesh(
        core_axis_name="c", subcore_axis_name="s",
        num_cores=NC, num_subcores=NS)

    @functools.partial(
        pl.kernel,
        out_type=jax.ShapeDtypeStruct((LPAD, D), jnp.float32),
        mesh=mesh,
        scratch_types=[
            pltpu.VMEM((PWPAD,), jnp.int32),
            pltpu.VMEM((NCHUNK, CHUNK), jnp.int32),
            pltpu.VMEM((PWPAD, D), jnp.float32),
            pltpu.VMEM((ZROWS, D), jnp.float32),
            pltpu.VMEM_SHARED((RANGE + DUMP, D), jnp.float32),
            pltpu.SemaphoreType.DMA,
        ],
        compiler_params=pltpu.CompilerParams(use_tc_tiling_on_sc=False),
    )
    def k(x_hbm, lab_hbm, out_hbm, lab_v, idx_v, x_v, z_v, s_sh, sem):
        c = lax.axis_index("c")
        s = lax.axis_index("s")
        zvec = jnp.zeros((D,), jnp.float32)

        @pl.loop(0, ZROWS)
        def _zfill(i):
            z_v[i, :] = zvec

        pltpu.sync_copy(lab_hbm.at[s], lab_v)
        pltpu.sync_copy(x_hbm.at[pl.ds(s * PW, PW)], x_v.at[pl.ds(0, PW)])

        lane = lax.iota(jnp.int32, NLANE)

        for p in range(NPASS):
            blk = c * NPASS + p
            base = blk * RANGE
            for z in range(STRIPE // ZROWS):
                pltpu.sync_copy(
                    z_v, s_sh.at[pl.ds(s * STRIPE + z * ZROWS, ZROWS)])

            @pl.loop(0, NCHUNK)
            def _mkidx(j):
                for t in range(CHUNK // NLANE):
                    lab = lab_v[pl.ds(j * CHUNK + t * NLANE, NLANE)]
                    rel = lab - base
                    inr = (rel >= 0) & (rel < RANGE)
                    dump = (RANGE + t * NLANE) + lane
                    idx_v[j, pl.ds(t * NLANE, NLANE)] = jnp.where(
                        inr, rel, dump)

            plsc.subcore_barrier()

            for g in range(NGROUP):
                @pl.loop(g * GSZ, (g + 1) * GSZ)
                def _fire(j):
                    pltpu.async_copy(x_v.at[pl.ds(j * CHUNK, CHUNK)],
                                     s_sh.at[idx_v.at[j]], sem, add=True)

                @pl.loop(g * GSZ, (g + 1) * GSZ)
                def _drain(j):
                    pltpu.make_async_copy(x_v.at[pl.ds(j * CHUNK, CHUNK)],
                                          s_sh.at[idx_v.at[j]], sem).wait()

            plsc.subcore_barrier()
            pltpu.sync_copy(
                s_sh.at[pl.ds(s * STRIPE, STRIPE)],
                out_hbm.at[pl.ds(base + s * STRIPE, STRIPE)])

    return k(x_w, lab_w)


def _tc_matmul_body(w_ref, s_ref, o_ref):
    kstep = pl.program_id(0)

    @pl.when(kstep == 0)
    def _():
        o_ref[...] = jnp.zeros_like(o_ref)

    w = w_ref[...]
    col = lax.broadcasted_iota(jnp.int32, (1, LB), 1) + kstep * LB
    w = jnp.where(col < L, w, 0.0)
    o_ref[...] += jnp.dot(w, s_ref[...], preferred_element_type=jnp.float32)


def kernel(x, node_labels, Param_W, Param_b):
    x = x.astype(jnp.float32)
    labels = node_labels.astype(jnp.int32)

    lab_w = jnp.pad(labels.reshape(NS, PW), ((0, 0), (0, PWPAD - PW)),
                    constant_values=-1)
    x_w = x

    s_tab = _sc_segment_sum(x_w, lab_w)

    w2 = Param_W.reshape(O, L)

    out = pl.pallas_call(
        _tc_matmul_body,
        grid=(KSTEPS,),
        in_specs=[
            pl.BlockSpec((O, LB), lambda k: (0, k)),
            pl.BlockSpec((LB, D), lambda k: (k, 0)),
        ],
        out_specs=pl.BlockSpec((O, D), lambda k: (0, 0)),
        out_shape=jax.ShapeDtypeStruct((O, D), jnp.float32),
        compiler_params=pltpu.CompilerParams(
            dimension_semantics=("arbitrary",)),
    )(w2, s_tab)

    return (out + Param_b.reshape(O, D)).reshape(C, O, D)
